# Initial kernel scaffold; baseline (speedup 1.0000x reference)
import jax
import jax.numpy as jnp
from jax.experimental import pallas as pl


def kernel(pred, target):
    raise NotImplementedError("write your pallas kernel here")


# speedup vs baseline: 22.0587x; 22.0587x over previous
import jax
import jax.numpy as jnp
from jax import lax
from jax.experimental import pallas as pl
from jax.experimental.pallas import tpu as pltpu

_ALPHA = 0.75
_GAMMA = 2.0
_DICE_WEIGHT = 0.5
_OHEM_RATIO = 0.25
_SMOOTH = 1e-06

_ROWS = 4096
_COLS = 1024
_N = _ROWS * _COLS
_K = _N // 4
_CHUNK = 512
_NCHUNK = _ROWS // _CHUNK


def _loss_kernel(pred_ref, target_ref, out_ref, bits_ref, acc_ref):
    i = pl.program_id(0)

    p = pred_ref[...]
    t = target_ref[...]

    bce = jnp.maximum(p, 0.0) - p * t + jnp.log1p(jnp.exp(-jnp.abs(p)))
    pt = jnp.exp(-bce)
    alpha_t = t * _ALPHA + (1.0 - t) * (1.0 - _ALPHA)
    om = 1.0 - pt
    focal = alpha_t * (om * om) * bce

    bits_ref[pl.ds(i * _CHUNK, _CHUNK), :] = lax.bitcast_convert_type(
        focal, jnp.int32)

    prob = jax.nn.sigmoid(p)
    s_prob = jnp.sum(prob)
    s_tgt = jnp.sum(t)
    s_int = jnp.sum(prob * t)

    @pl.when(i == 0)
    def _():
        acc_ref[0] = s_prob
        acc_ref[1] = s_tgt
        acc_ref[2] = s_int

    @pl.when(i > 0)
    def _():
        acc_ref[0] += s_prob
        acc_ref[1] += s_tgt
        acc_ref[2] += s_int

    @pl.when(i == _NCHUNK - 1)
    def _():
        def body(_, carry):
            lo, hi = carry
            mid = lo + (hi - lo + 1) // 2
            cnt = jnp.sum((bits_ref[...] >= mid).astype(jnp.int32))
            big = cnt >= _K
            new_lo = jnp.where(big, mid, lo)
            new_hi = jnp.where(big, hi, mid - 1)
            return new_lo, new_hi

        lo0 = jnp.int32(0)
        hi0 = jnp.int32(0x7F7FFFFF)
        t_bits, _ = lax.fori_loop(0, 31, body, (lo0, hi0))

        bits = bits_ref[...]
        gt = bits > t_bits
        cnt_gt = jnp.sum(gt.astype(jnp.int32))
        vals = lax.bitcast_convert_type(bits, jnp.float32)
        sum_gt = jnp.sum(jnp.where(gt, vals, 0.0))
        t_val = lax.bitcast_convert_type(t_bits, jnp.float32)
        focal_loss = (sum_gt + t_val * (_K - cnt_gt).astype(jnp.float32)) / _K

        dice_loss = 1.0 - (2.0 * acc_ref[2] + _SMOOTH) / (
            acc_ref[0] + acc_ref[1] + _SMOOTH)
        out_ref[0] = _DICE_WEIGHT * dice_loss + (1.0 - _DICE_WEIGHT) * focal_loss


def kernel(pred, target):
    p2 = pred.reshape(_ROWS, _COLS)
    t2 = target.reshape(_ROWS, _COLS)
    out = pl.pallas_call(
        _loss_kernel,
        grid=(_NCHUNK,),
        in_specs=[
            pl.BlockSpec((_CHUNK, _COLS), lambda i: (i, 0)),
            pl.BlockSpec((_CHUNK, _COLS), lambda i: (i, 0)),
        ],
        out_specs=pl.BlockSpec(memory_space=pltpu.SMEM),
        out_shape=jax.ShapeDtypeStruct((1,), jnp.float32),
        scratch_shapes=[
            pltpu.VMEM((_ROWS, _COLS), jnp.int32),
            pltpu.SMEM((3,), jnp.float32),
        ],
    )(p2, t2)
    return out[0]
